# baseline (speedup 1.0000x reference)
import jax
import jax.numpy as jnp
import numpy as np
from jax import lax
from jax.experimental import pallas as pl
from jax.experimental.pallas import tpu as pltpu
from jax.experimental.pallas import tpu_sc as plsc

jax.config.update("jax_enable_x64", True)

B = 16384
KD = 64
ROWW = 128
NC = 2
NS = 16
L = 16
NW = NC * NS
BPW = B // NW
CB = 128
NCHUNK = BPW // CB
TB = 1024
TG = B // TB


def _sc_gather_body(ih, t, out, idx_v, r0, r1, s0, s1):
    wid = lax.axis_index("s") * np.int32(NC) + lax.axis_index("c")
    base = wid * np.int32(BPW)
    rws = (r0, r1)
    sms = (s0, s1)

    def start_chunk(c, buf):
        off = base + np.int32(c * CB)
        pltpu.sync_copy(ih.at[pl.ds(off, CB)], idx_v)
        return pltpu.async_copy(t.at[idx_v], rws[buf], sms[buf])

    cp = start_chunk(0, 0)
    for c in range(NCHUNK):
        cp.wait()
        if c + 1 < NCHUNK:
            nxt = start_chunk(c + 1, (c + 1) % 2)
        off = base + np.int32(c * CB)
        pltpu.sync_copy(rws[c % 2], out.at[pl.ds(off, CB)])
        if c + 1 < NCHUNK:
            cp = nxt


def _sc_gather(ih, t):
    mesh = plsc.VectorSubcoreMesh(core_axis_name="c", subcore_axis_name="s")
    return pl.kernel(
        _sc_gather_body,
        out_type=jax.ShapeDtypeStruct((B, ROWW), jnp.float32),
        mesh=mesh,
        compiler_params=pltpu.CompilerParams(needs_layout_passes=False),
        scratch_types=[
            pltpu.VMEM((CB,), jnp.int32),
            pltpu.VMEM((CB, ROWW), jnp.float32),
            pltpu.VMEM((CB, ROWW), jnp.float32),
            pltpu.SemaphoreType.DMA,
            pltpu.SemaphoreType.DMA,
        ],
    )(ih, t)


def _tc_combine_body(p0_ref, p1_ref, p2_ref, g0_ref, g1_ref, g2_ref, out_ref):
    def half(g_ref, p_ref):
        sel = p_ref[...] > 0
        lo = g_ref[:, :KD]
        hi = g_ref[:, KD:]
        return jnp.where(sel, hi, lo)

    prod = (half(g0_ref, p0_ref) * half(g1_ref, p1_ref)
            * half(g2_ref, p2_ref))
    out_ref[...] = jnp.sum(prod, axis=1)


def _tc_combine(p0, p1, p2, g0, g1, g2):
    blk = lambda i: (i, np.int32(0))
    return pl.pallas_call(
        _tc_combine_body,
        out_shape=jax.ShapeDtypeStruct((B,), jnp.float32),
        grid=(TG,),
        in_specs=[
            pl.BlockSpec((TB, 1), blk),
            pl.BlockSpec((TB, 1), blk),
            pl.BlockSpec((TB, 1), blk),
            pl.BlockSpec((TB, ROWW), blk),
            pl.BlockSpec((TB, ROWW), blk),
            pl.BlockSpec((TB, ROWW), blk),
        ],
        out_specs=pl.BlockSpec((TB,), lambda i: (i,)),
    )(p0, p1, p2, g0, g1, g2)


def kernel(indices, f0, f1, f2, log_sigma):
    idx = indices.astype(jnp.int32)
    ih = idx >> 1
    pb = idx & 1
    t0 = f0.astype(jnp.float32).reshape(f0.shape[0] // 2, ROWW)
    t1 = f1.astype(jnp.float32).reshape(f1.shape[0] // 2, ROWW)
    t2 = f2.astype(jnp.float32).reshape(f2.shape[0] // 2, ROWW)
    g0 = _sc_gather(ih[:, 0], t0)
    g1 = _sc_gather(ih[:, 1], t1)
    g2 = _sc_gather(ih[:, 2], t2)
    res32 = _tc_combine(pb[:, 0:1], pb[:, 1:2], pb[:, 2:3], g0, g1, g2)
    return (res32.astype(jnp.float64), jnp.clip(log_sigma, -2.5, 0.0))


# speedup vs baseline: 1.1023x; 1.1023x over previous
import jax
import jax.numpy as jnp
import numpy as np
from jax import lax
from jax.experimental import pallas as pl
from jax.experimental.pallas import tpu as pltpu
from jax.experimental.pallas import tpu_sc as plsc

jax.config.update("jax_enable_x64", True)

B = 16384
KD = 64
ROWW = 128
NC = 2
NS = 16
L = 16
NW = NC * NS
BPW = B // NW
CB = 128
NCHUNK = BPW // CB
GPC = CB // L


def _sc_body(ih0, ih1, ih2, pb0, pb1, pb2, t0, t1, t2, out,
             idx_v,
             pv00, pv01, pv02, pv10, pv11, pv12,
             r00, r01, r02, r10, r11, r12,
             out_v,
             s00, s01, s02, s10, s11, s12):
    wid = lax.axis_index("s") * np.int32(NC) + lax.axis_index("c")
    base = wid * np.int32(BPW)
    iot = lax.iota(jnp.int32, L)
    pvs = ((pv00, pv01, pv02), (pv10, pv11, pv12))
    rws = ((r00, r01, r02), (r10, r11, r12))
    sms = ((s00, s01, s02), (s10, s11, s12))

    def start_chunk(c, buf):
        off = base + np.int32(c * CB)
        pltpu.sync_copy(ih0.at[pl.ds(off, CB)], idx_v.at[np.int32(0)])
        pltpu.sync_copy(ih1.at[pl.ds(off, CB)], idx_v.at[np.int32(1)])
        pltpu.sync_copy(ih2.at[pl.ds(off, CB)], idx_v.at[np.int32(2)])
        pltpu.sync_copy(pb0.at[pl.ds(off, CB)], pvs[buf][0])
        pltpu.sync_copy(pb1.at[pl.ds(off, CB)], pvs[buf][1])
        pltpu.sync_copy(pb2.at[pl.ds(off, CB)], pvs[buf][2])
        return (
            pltpu.async_copy(t0.at[idx_v.at[np.int32(0)]], rws[buf][0],
                             sms[buf][0]),
            pltpu.async_copy(t1.at[idx_v.at[np.int32(1)]], rws[buf][1],
                             sms[buf][1]),
            pltpu.async_copy(t2.at[idx_v.at[np.int32(2)]], rws[buf][2],
                             sms[buf][2]),
        )

    def compute_chunk(c, buf):
        r0_v, r1_v, r2_v = rws[buf]
        pb0_v, pb1_v, pb2_v = pvs[buf]

        def group_body(g, carry):
            offs, loc = carry
            row_idx = jnp.broadcast_to(loc, (L,)) + iot
            pv0 = pb0_v[pl.ds(loc, L)]
            pv1 = pb1_v[pl.ds(loc, L)]
            pv2 = pb2_v[pl.ds(loc, L)]
            half = np.int32(KD // 2)

            def k_body(k, kcarry):
                acc_a, acc_b, c0, c1, c2 = kcarry
                v0 = plsc.load_gather(r0_v, [row_idx, c0])
                v1 = plsc.load_gather(r1_v, [row_idx, c1])
                v2 = plsc.load_gather(r2_v, [row_idx, c2])
                w0 = plsc.load_gather(r0_v, [row_idx, c0 + half])
                w1 = plsc.load_gather(r1_v, [row_idx, c1 + half])
                w2 = plsc.load_gather(r2_v, [row_idx, c2 + half])
                return (acc_a + v0 * v1 * v2, acc_b + w0 * w1 * w2,
                        c0 + np.int32(1), c1 + np.int32(1), c2 + np.int32(1))

            acc_a, acc_b, _, _, _ = lax.fori_loop(
                np.int32(0), half, k_body,
                (jnp.zeros((L,), jnp.float32), jnp.zeros((L,), jnp.float32),
                 pv0, pv1, pv2),
                unroll=8)
            out_v[pl.ds(offs, L)] = acc_a + acc_b
            return (offs + np.int32(L), loc + np.int32(L))

        lax.fori_loop(np.int32(0), np.int32(GPC), group_body,
                      (jnp.full((), c * CB, jnp.int32),
                       jnp.full((), 0, jnp.int32)))

    cps = start_chunk(0, 0)
    for c in range(NCHUNK):
        for cp in cps:
            cp.wait()
        if c + 1 < NCHUNK:
            nxt = start_chunk(c + 1, (c + 1) % 2)
        compute_chunk(c, c % 2)
        if c + 1 < NCHUNK:
            cps = nxt
    pltpu.sync_copy(out_v, out.at[pl.ds(base, BPW)])


def _sc_call(ih0, ih1, ih2, pb0, pb1, pb2, t0, t1, t2):
    mesh = plsc.VectorSubcoreMesh(core_axis_name="c", subcore_axis_name="s")
    return pl.kernel(
        _sc_body,
        out_type=jax.ShapeDtypeStruct((B,), jnp.float32),
        mesh=mesh,
        compiler_params=pltpu.CompilerParams(needs_layout_passes=False),
        scratch_types=(
            [pltpu.VMEM((3, CB), jnp.int32)]
            + [pltpu.VMEM((CB,), jnp.int32) for _ in range(6)]
            + [pltpu.VMEM((CB, ROWW), jnp.float32) for _ in range(6)]
            + [pltpu.VMEM((BPW,), jnp.float32)]
            + [pltpu.SemaphoreType.DMA for _ in range(6)]
        ),
    )(ih0, ih1, ih2, pb0, pb1, pb2, t0, t1, t2)


def kernel(indices, f0, f1, f2, log_sigma):
    idx = indices.astype(jnp.int32)
    ih = idx >> 1
    pb = (idx & 1) << 6
    t0 = f0.reshape(f0.shape[0] // 2, ROWW).astype(jnp.float32)
    t1 = f1.reshape(f1.shape[0] // 2, ROWW).astype(jnp.float32)
    t2 = f2.reshape(f2.shape[0] // 2, ROWW).astype(jnp.float32)
    res32 = _sc_call(ih[:, 0], ih[:, 1], ih[:, 2],
                     pb[:, 0], pb[:, 1], pb[:, 2], t0, t1, t2)
    return (res32.astype(jnp.float64), jnp.clip(log_sigma, -2.5, 0.0))
